# Initial kernel scaffold; baseline (speedup 1.0000x reference)
import jax
import jax.numpy as jnp
from jax.experimental import pallas as pl


def kernel(s, v, edge_index_local, d_local, r_local, e_local, edge_index_global, d_global, r_global, e_global, batch, params):
    raise NotImplementedError("write your pallas kernel here")


# speedup vs baseline: 24.2018x; 24.2018x over previous
import functools

import jax
import jax.numpy as jnp
from jax import lax
from jax.experimental import pallas as pl
from jax.experimental.pallas import tpu as pltpu
from jax.experimental.pallas import tpu_sc as plsc

_SDIM, _VDIM, _EDIM, _RBF = 64, 16, 16, 64
_CUT = 5.0
_NG = 32
_N = 1024
_EB = 512
_NW = 32
_TYPES = ['g', 'r', 'r', 'g', 'r']
_HAS_V = [False, True, True, True, True]
_USE_MLP = [True, True, True, True, False]


def _silu(x):
    return x / (1.0 + jnp.exp(-x))


_PREC = lax.Precision.HIGHEST


def _dot(a, b):
    return lax.dot_general(a, b, (((1,), (0,)), ((), ())), precision=_PREC)


def _dott(a, b):
    return lax.dot_general(a, b, (((0,), (0,)), ((), ())), precision=_PREC)


def _dot_bf(a, b):
    return lax.dot_general(a.astype(jnp.bfloat16), b.astype(jnp.bfloat16),
                           (((1,), (0,)), ((), ())),
                           preferred_element_type=jnp.float32)


def _bf(x):
    return x.astype(jnp.bfloat16).astype(jnp.float32)


def _ln_body(s_ref, v_ref, oh_ref, lnw_ref, lnb_ref, sn_ref, vn_ref):
    s = s_ref[...]
    v = v_ref[...]
    oh = oh_ref[...]
    ones = jnp.ones((s.shape[0], 1), jnp.float32)
    cnt = jnp.maximum(_dott(oh, ones), 1.0)
    mean = _dott(oh, s) / cnt
    scent = s - _dot(oh, mean)
    var = _dott(oh, scent * scent) / cnt
    rstd = lax.rsqrt(var + 1e-6)
    sn_ref[...] = lnw_ref[...] * scent * _dot(oh, rstd) + lnb_ref[...]
    vsq = v * v
    vn16 = vsq[:, 0:16] + vsq[:, 16:32] + vsq[:, 32:48]
    vmean = _dott(oh, vn16) / cnt
    vr = _dot(oh, lax.rsqrt(vmean + 1e-6))
    vn_ref[:, 0:16] = v[:, 0:16] * vr
    vn_ref[:, 16:32] = v[:, 16:32] * vr
    vn_ref[:, 32:48] = v[:, 32:48] * vr


def _ln_call(s, v, oh, lnw, lnb):
    return pl.pallas_call(
        _ln_body,
        out_shape=[jax.ShapeDtypeStruct((_N, _SDIM), jnp.float32),
                   jax.ShapeDtypeStruct((_N, 3 * _VDIM), jnp.float32)],
    )(s, v, oh, lnw.reshape(1, _SDIM), lnb.reshape(1, _SDIM))


def _make_conv_body(ltype, has_v):
    def body(*refs):
        if has_v:
            (ssrc_ref, sdst_ref, vsrc_ref, d_ref, r_ref, e_ref,
             w1a_ref, w1b_ref, w1c_ref, w1e_ref, b1_ref,
             w2s_ref, w2v1_ref, w2v2_ref, w2e_ref,
             b2s_ref, b2v1_ref, b2v2_ref, b2e_ref, cen_ref,
             msg_ref, enew_ref) = refs
        else:
            (ssrc_ref, sdst_ref, d_ref, r_ref, e_ref,
             w1a_ref, w1b_ref, w1c_ref, w1e_ref, b1_ref,
             w2s_ref, w2v1_ref, w2v2_ref, w2e_ref,
             b2s_ref, b2v1_ref, b2v2_ref, b2e_ref, cen_ref,
             msg_ref, enew_ref) = refs
        ssrc = ssrc_ref[...]
        sdst = sdst_ref[...]
        e = e_ref[...]
        d = d_ref[...]
        if ltype == 'r':
            cen = cen_ref[...]
            rbf = jnp.exp(-((d - cen) ** 2) * (_RBF / _CUT))
            dc = jnp.clip(d, 0.0, _CUT)
            fc = 0.5 * (jnp.cos((jnp.pi / _CUT) * dc) + 1.0)
            hc = _dot_bf(rbf * fc, w1c_ref[...])
        else:
            hc = _bf(d) * _bf(w1c_ref[...])
        h1 = _silu(_dot_bf(ssrc, w1a_ref[...]) + _dot_bf(sdst, w1b_ref[...])
                   + hc + _dot_bf(e, w1e_ref[...]) + b1_ref[...])
        gs = _dot_bf(h1, w2s_ref[...]) + b2s_ref[...]
        gv1 = _dot_bf(h1, w2v1_ref[...]) + b2v1_ref[...]
        gv2 = _dot_bf(h1, w2v2_ref[...]) + b2v2_ref[...]
        enew_ref[...] = _dot_bf(h1, w2e_ref[...]) + b2e_ref[...]
        r = r_ref[...]
        parts = [_silu(gs) * ssrc]
        for a in range(3):
            mv = r[:, a:a + 1] * gv1
            if has_v:
                mv = mv + vsrc_ref[:, 16 * a:16 * a + 16] * gv2
            parts.append(mv)
        msg_ref[...] = jnp.concatenate(parts, axis=1)
    return body


def _conv_call(ltype, has_v, ne, ssrc, sdst, vsrc, d, r, e, w):
    nb = ne // _EB
    eb = pl.BlockSpec((_EB, None), lambda i: (i, 0))

    def full(x):
        return pl.BlockSpec(x.shape, lambda i: tuple(0 for _ in x.shape))

    edge_ins = [ssrc, sdst] + ([vsrc] if has_v else []) + [d, r, e]
    wlist = [w['w1a'], w['w1b'], w['w1c'], w['w1e'], w['b1'],
             w['w2s'], w['w2v1'], w['w2v2'], w['w2e'],
             w['b2s'], w['b2v1'], w['b2v2'], w['b2e'], w['cen']]
    in_specs = ([pl.BlockSpec((_EB, x.shape[1]), lambda i: (i, 0)) for x in edge_ins]
                + [full(x) for x in wlist])
    del eb
    return pl.pallas_call(
        _make_conv_body(ltype, has_v),
        grid=(nb,),
        in_specs=in_specs,
        out_specs=[pl.BlockSpec((_EB, 112), lambda i: (i, 0)),
                   pl.BlockSpec((_EB, _EDIM), lambda i: (i, 0))],
        out_shape=[jax.ShapeDtypeStruct((ne, 112), jnp.float32),
                   jax.ShapeDtypeStruct((ne, _EDIM), jnp.float32)],
    )(*edge_ins, *wlist)


def _make_node_body(use_mlp):
    def body(*refs):
        if use_mlp:
            (sn_ref, vn_ref, a0_ref, a1_ref, c0_ref, c1_ref,
             wn1a_ref, wn1b_ref, wn1c_ref, bn1_ref,
             wn2s_ref, wn2v_ref, bn2s_ref, bn2v_ref,
             s_ref, v_ref) = refs
        else:
            (sn_ref, vn_ref, a0_ref, a1_ref, c0_ref, c1_ref,
             s_ref, v_ref) = refs
        sn = sn_ref[...]
        vn = vn_ref[...]
        a = a0_ref[...] + a1_ref[...]
        cnt = jnp.maximum(c0_ref[...] + c1_ref[...], 1.0)
        s_agg = a[:, 0:64]
        va = [a[:, 64 + 16 * i:80 + 16 * i] / cnt for i in range(3)]
        if use_mlp:
            vnorm = jnp.sqrt(va[0] * va[0] + va[1] * va[1] + va[2] * va[2]
                             + 1e-6)
            u = _silu(_dot_bf(sn, wn1a_ref[...]) + _dot_bf(s_agg, wn1b_ref[...])
                      + _dot_bf(vnorm, wn1c_ref[...]) + bn1_ref[...])
            s_ref[...] = sn + _dot_bf(u, wn2s_ref[...]) + bn2s_ref[...]
            uv = _dot_bf(u, wn2v_ref[...]) + bn2v_ref[...]
            for i in range(3):
                v_ref[:, 16 * i:16 * i + 16] = (vn[:, 16 * i:16 * i + 16]
                                                + va[i] * uv)
        else:
            s_ref[...] = sn + s_agg
            for i in range(3):
                v_ref[:, 16 * i:16 * i + 16] = vn[:, 16 * i:16 * i + 16] + va[i]
    return body


def _node_call(use_mlp, sn, vn, acc, cacc, w):
    ins = [sn, vn, acc[0], acc[1], cacc[0], cacc[1]]
    if use_mlp:
        ins += [w['wn1a'], w['wn1b'], w['wn1c'], w['bn1'],
                w['wn2s'], w['wn2v'], w['bn2s'], w['bn2v']]
    return pl.pallas_call(
        _make_node_body(use_mlp),
        out_shape=[jax.ShapeDtypeStruct((_N, _SDIM), jnp.float32),
                   jax.ShapeDtypeStruct((_N, 3 * _VDIM), jnp.float32)],
    )(*ins)


@functools.lru_cache(maxsize=None)
def _make_multigather(ne, ds, tabns):
    nt = len(ds)
    cpw = ne // _NW
    c = min(cpw, 512)
    nch = cpw // c
    mesh = plsc.VectorSubcoreMesh(core_axis_name="c", subcore_axis_name="s")
    out_type = [jax.ShapeDtypeStruct((ne, d), jnp.float32) for d in ds]
    scratch = ([pltpu.VMEM((c,), jnp.int32) for _ in range(nt)]
               + [pltpu.VMEM((c, d), jnp.float32) for d in ds]
               + [pltpu.SemaphoreType.DMA for _ in range(nt)])

    @functools.partial(pl.kernel, out_type=out_type, mesh=mesh,
                       scratch_types=scratch,
                       compiler_params=pltpu.CompilerParams(
                           use_tc_tiling_on_sc=False))
    def k(*refs):
        tabs = refs[0:nt]
        idxs = refs[nt:2 * nt]
        outs = refs[2 * nt:3 * nt]
        idx_v = refs[3 * nt:4 * nt]
        row_v = refs[4 * nt:5 * nt]
        sems = refs[5 * nt:6 * nt]
        wid = lax.axis_index("s") * 2 + lax.axis_index("c")
        base0 = wid * cpw

        def chunk(ci, _):
            base = base0 + ci * c
            for t in range(nt):
                pltpu.sync_copy(idxs[t].at[pl.ds(base, c)], idx_v[t])
            cps = [pltpu.async_copy(tabs[t].at[idx_v[t]], row_v[t], sems[t])
                   for t in range(nt)]
            for cp in cps:
                cp.wait()
            for t in range(nt):
                pltpu.sync_copy(row_v[t], outs[t].at[pl.ds(base, c)])
            return _

        if nch == 1:
            chunk(0, 0)
        else:
            lax.fori_loop(0, nch, chunk, 0)

    return k


def _gather(tables, idxs):
    ne = idxs[0].shape[0]
    ds = tuple(int(t.shape[1]) for t in tables)
    tabns = tuple(int(t.shape[0]) for t in tables)
    k = _make_multigather(ne, ds, tabns)
    return k(*tables, *idxs)


@functools.lru_cache(maxsize=None)
def _make_scatter(ne, d):
    cpw = ne // _NW
    c = min(cpw, 512)
    nch = cpw // c
    mesh = plsc.VectorSubcoreMesh(core_axis_name="c", subcore_axis_name="s")
    out_type = jax.ShapeDtypeStruct((2, _N, d), jnp.float32)
    scratch = [pltpu.VMEM((c,), jnp.int32),
               pltpu.VMEM((c, d), jnp.float32),
               pltpu.VMEM_SHARED((_N, d), jnp.float32),
               pltpu.SemaphoreType.DMA]

    @functools.partial(pl.kernel, out_type=out_type, mesh=mesh,
                       scratch_types=scratch,
                       compiler_params=pltpu.CompilerParams(
                           use_tc_tiling_on_sc=False))
    def k(msg_hbm, dst_hbm, zero_hbm, out_hbm, idx_v, rows_v, acc_sh, sem):
        cid = lax.axis_index("c")
        sid = lax.axis_index("s")
        wid = sid * 2 + cid

        @pl.when(sid == 0)
        def _():
            pltpu.sync_copy(zero_hbm, acc_sh)

        plsc.subcore_barrier()

        def chunk(ci, _):
            base = wid * cpw + ci * c
            pltpu.sync_copy(dst_hbm.at[pl.ds(base, c)], idx_v)
            pltpu.sync_copy(msg_hbm.at[pl.ds(base, c)], rows_v)
            pltpu.sync_copy(rows_v, acc_sh.at[idx_v], add=True)
            return _

        if nch == 1:
            chunk(0, 0)
        else:
            lax.fori_loop(0, nch, chunk, 0)

        plsc.subcore_barrier()

        @pl.when(sid == 0)
        def _():
            pltpu.sync_copy(acc_sh, out_hbm.at[cid])

    return k


def _scatter(msg, dst, d):
    ne = msg.shape[0]
    zero = jnp.zeros((_N, d), jnp.float32)
    return _make_scatter(ne, d)(msg, dst, zero)


def _split_params(p, ltype, use_mlp):
    dflen = _RBF if ltype == 'r' else 1
    w1 = p['W1']
    w2 = p['W2']
    out = {
        'w1a': w1[0:64], 'w1b': w1[64:128],
        'w1c': w1[128:128 + dflen], 'w1e': w1[128 + dflen:],
        'b1': p['b1'].reshape(1, 64),
        'w2s': w2[:, 0:64], 'w2v1': w2[:, 64:80],
        'w2v2': w2[:, 80:96], 'w2e': w2[:, 96:112],
        'b2s': p['b2'][0:64].reshape(1, 64),
        'b2v1': p['b2'][64:80].reshape(1, 16),
        'b2v2': p['b2'][80:96].reshape(1, 16),
        'b2e': p['b2'][96:112].reshape(1, 16),
        'cen': jnp.linspace(0.0, _CUT, _RBF).reshape(1, _RBF),
    }
    if use_mlp:
        wn1 = p['Wn1']
        wn2 = p['Wn2']
        out.update({
            'wn1a': wn1[0:64], 'wn1b': wn1[64:128], 'wn1c': wn1[128:144],
            'bn1': p['bn1'].reshape(1, 64),
            'wn2s': wn2[:, 0:64], 'wn2v': wn2[:, 64:80],
            'bn2s': p['bn2'][0:64].reshape(1, 64),
            'bn2v': p['bn2'][64:80].reshape(1, 16),
        })
    return out


def kernel(s, v, edge_index_local, d_local, r_local, e_local,
           edge_index_global, d_global, r_global, e_global, batch, params):
    n = s.shape[0]
    el = edge_index_local.astype(jnp.int32)
    eg = edge_index_global.astype(jnp.int32)
    nl, ng = el.shape[1], eg.shape[1]
    src_l, dst_l = el[0], el[1]
    src_g, dst_g = eg[0], eg[1]

    kl = src_l * n + dst_l
    kg = src_g * n + dst_g
    map_g = jnp.full((n * n,), -1, jnp.int32).at[kg].max(
        jnp.arange(ng, dtype=jnp.int32))
    map_l = jnp.full((n * n,), -1, jnp.int32).at[kl].max(
        jnp.arange(nl, dtype=jnp.int32))
    g2l = map_g[kl]
    g2g = map_g[kg]
    l2g = map_l[kg]
    l2l = map_l[kl]
    idx_l1 = jnp.where(g2l >= 0, g2l, ng)
    idx_g3 = jnp.where(l2g >= 0, l2g, nl + g2g)
    idx_l4 = jnp.where(g2l >= 0, g2l, ng + l2l)

    oh = (batch[:, None] == jnp.arange(_NG)[None, :]).astype(jnp.float32)
    vflat = v.reshape(n, 3 * _VDIM)
    d_l2 = d_local[:, None]
    d_g2 = d_global[:, None]
    wts = [_split_params(params[i], _TYPES[i], _USE_MLP[i]) for i in range(5)]

    cacc_l = _scatter(jnp.ones((nl, 16), jnp.float32), dst_l, 16)
    cacc_g = _scatter(jnp.ones((ng, 16), jnp.float32), dst_g, 16)

    def run_layer(i, scur, vcur, ein, etab, eidx):
        is_g = _TYPES[i] == 'g'
        src, dst = (src_g, dst_g) if is_g else (src_l, dst_l)
        d2, r = (d_g2, r_global) if is_g else (d_l2, r_local)
        ne = ng if is_g else nl
        cacc = cacc_g if is_g else cacc_l
        sn, vn = _ln_call(scur, vcur, oh, params[i]['ln_w'], params[i]['ln_b'])
        tabs = [sn, sn] + ([vn] if _HAS_V[i] else [])
        idxs = [src, dst] + ([src] if _HAS_V[i] else [])
        if etab is not None:
            tabs.append(etab)
            idxs.append(eidx)
        g = _gather(tuple(tabs), tuple(idxs))
        ssrc, sdst = g[0], g[1]
        vsrc = g[2] if _HAS_V[i] else None
        if etab is not None:
            ein = g[-1]
        msg, enew = _conv_call(_TYPES[i], _HAS_V[i], ne,
                               ssrc, sdst, vsrc, d2, r, ein, wts[i])
        acc = _scatter(msg, dst, 112)
        s2, v2 = _node_call(_USE_MLP[i], sn, vn, acc, cacc, wts[i])
        return s2, v2, enew

    zrow = jnp.zeros((1, _EDIM), jnp.float32)
    s0, v0, eg0 = run_layer(0, s, vflat, e_global, None, None)
    eg0r = _bf(eg0)
    s1, v1, el1 = run_layer(1, s0, v0, None,
                            jnp.concatenate([eg0r, zrow], 0), idx_l1)
    s2, v2, el2 = run_layer(2, s1, v1, el1, None, None)
    el2r = _bf(el2)
    s3, v3, eg3 = run_layer(3, s2, v2, None,
                            jnp.concatenate([el2r, eg0r], 0), idx_g3)
    eg3r = _bf(eg3)
    s4, v4, el4 = run_layer(4, s3, v3, None,
                            jnp.concatenate([eg3r, el2r], 0), idx_l4)
    (e_out,) = _gather((jnp.concatenate([_bf(el4), eg3r], 0),), (idx_g3,))
    return s4, v4.reshape(n, 3, _VDIM), e_out


# speedup vs baseline: 24.3210x; 1.0049x over previous
import functools

import jax
import jax.numpy as jnp
from jax import lax
from jax.experimental import pallas as pl
from jax.experimental.pallas import tpu as pltpu
from jax.experimental.pallas import tpu_sc as plsc

_SDIM, _VDIM, _EDIM, _RBF = 64, 16, 16, 64
_CUT = 5.0
_NG = 32
_N = 1024
_EB = 512
_NW = 32
_TYPES = ['g', 'r', 'r', 'g', 'r']
_HAS_V = [False, True, True, True, True]
_USE_MLP = [True, True, True, True, False]


def _silu(x):
    return x / (1.0 + jnp.exp(-x))


_PREC = lax.Precision.HIGHEST


def _dot(a, b):
    return lax.dot_general(a, b, (((1,), (0,)), ((), ())), precision=_PREC)


def _dott(a, b):
    return lax.dot_general(a, b, (((0,), (0,)), ((), ())), precision=_PREC)


def _dot_bf(a, b):
    return lax.dot_general(a.astype(jnp.bfloat16), b.astype(jnp.bfloat16),
                           (((1,), (0,)), ((), ())),
                           preferred_element_type=jnp.float32)


def _bf(x):
    return x.astype(jnp.bfloat16).astype(jnp.float32)


def _ln_math(s, v, oh, lnw, lnb):
    ones = jnp.ones((s.shape[0], 1), jnp.float32)
    cnt = jnp.maximum(_dott(oh, ones), 1.0)
    mean = _dott(oh, s) / cnt
    scent = s - _dot(oh, mean)
    var = _dott(oh, scent * scent) / cnt
    rstd = lax.rsqrt(var + 1e-6)
    sn = lnw * scent * _dot(oh, rstd) + lnb
    vsq = v * v
    vn16 = vsq[:, 0:16] + vsq[:, 16:32] + vsq[:, 32:48]
    vmean = _dott(oh, vn16) / cnt
    vr = _dot(oh, lax.rsqrt(vmean + 1e-6))
    vn = jnp.concatenate([v[:, 0:16] * vr, v[:, 16:32] * vr,
                          v[:, 32:48] * vr], axis=1)
    return sn, vn


def _ln_body(s_ref, v_ref, oh_ref, lnw_ref, lnb_ref, sn_ref, vn_ref):
    sn, vn = _ln_math(s_ref[...], v_ref[...], oh_ref[...],
                      lnw_ref[...], lnb_ref[...])
    sn_ref[...] = sn
    vn_ref[...] = vn


def _ln_call(s, v, oh, lnw, lnb):
    return pl.pallas_call(
        _ln_body,
        out_shape=[jax.ShapeDtypeStruct((_N, _SDIM), jnp.float32),
                   jax.ShapeDtypeStruct((_N, 3 * _VDIM), jnp.float32)],
    )(s, v, oh, lnw.reshape(1, _SDIM), lnb.reshape(1, _SDIM))


def _make_conv_body(ltype, has_v):
    def body(*refs):
        if has_v:
            (ssrc_ref, sdst_ref, vsrc_ref, d_ref, r_ref, e_ref,
             w1a_ref, w1b_ref, w1c_ref, w1e_ref, b1_ref,
             w2s_ref, w2v1_ref, w2v2_ref, w2e_ref,
             b2s_ref, b2v1_ref, b2v2_ref, b2e_ref, cen_ref,
             msg_ref, enew_ref) = refs
        else:
            (ssrc_ref, sdst_ref, d_ref, r_ref, e_ref,
             w1a_ref, w1b_ref, w1c_ref, w1e_ref, b1_ref,
             w2s_ref, w2v1_ref, w2v2_ref, w2e_ref,
             b2s_ref, b2v1_ref, b2v2_ref, b2e_ref, cen_ref,
             msg_ref, enew_ref) = refs
        ssrc = ssrc_ref[...]
        sdst = sdst_ref[...]
        e = e_ref[...]
        d = d_ref[...]
        if ltype == 'r':
            cen = cen_ref[...]
            rbf = jnp.exp(-((d - cen) ** 2) * (_RBF / _CUT))
            dc = jnp.clip(d, 0.0, _CUT)
            fc = 0.5 * (jnp.cos((jnp.pi / _CUT) * dc) + 1.0)
            hc = _dot_bf(rbf * fc, w1c_ref[...])
        else:
            hc = _bf(d) * _bf(w1c_ref[...])
        h1 = _silu(_dot_bf(ssrc, w1a_ref[...]) + _dot_bf(sdst, w1b_ref[...])
                   + hc + _dot_bf(e, w1e_ref[...]) + b1_ref[...])
        gs = _dot_bf(h1, w2s_ref[...]) + b2s_ref[...]
        gv1 = _dot_bf(h1, w2v1_ref[...]) + b2v1_ref[...]
        gv2 = _dot_bf(h1, w2v2_ref[...]) + b2v2_ref[...]
        enew_ref[...] = _dot_bf(h1, w2e_ref[...]) + b2e_ref[...]
        r = r_ref[...]
        parts = [_silu(gs) * ssrc]
        for a in range(3):
            mv = r[:, a:a + 1] * gv1
            if has_v:
                mv = mv + vsrc_ref[:, 16 * a:16 * a + 16] * gv2
            parts.append(mv)
        msg_ref[...] = jnp.concatenate(parts, axis=1)
    return body


def _conv_call(ltype, has_v, ne, ssrc, sdst, vsrc, d, r, e, w):
    nb = ne // _EB
    eb = pl.BlockSpec((_EB, None), lambda i: (i, 0))

    def full(x):
        return pl.BlockSpec(x.shape, lambda i: tuple(0 for _ in x.shape))

    edge_ins = [ssrc, sdst] + ([vsrc] if has_v else []) + [d, r, e]
    wlist = [w['w1a'], w['w1b'], w['w1c'], w['w1e'], w['b1'],
             w['w2s'], w['w2v1'], w['w2v2'], w['w2e'],
             w['b2s'], w['b2v1'], w['b2v2'], w['b2e'], w['cen']]
    in_specs = ([pl.BlockSpec((_EB, x.shape[1]), lambda i: (i, 0)) for x in edge_ins]
                + [full(x) for x in wlist])
    del eb
    return pl.pallas_call(
        _make_conv_body(ltype, has_v),
        grid=(nb,),
        in_specs=in_specs,
        out_specs=[pl.BlockSpec((_EB, 112), lambda i: (i, 0)),
                   pl.BlockSpec((_EB, _EDIM), lambda i: (i, 0))],
        out_shape=[jax.ShapeDtypeStruct((ne, 112), jnp.float32),
                   jax.ShapeDtypeStruct((ne, _EDIM), jnp.float32)],
    )(*edge_ins, *wlist)


def _make_node_body(use_mlp, fuse_ln):
    def body(*refs):
        if use_mlp:
            (sn_ref, vn_ref, a0_ref, a1_ref, c0_ref, c1_ref,
             wn1a_ref, wn1b_ref, wn1c_ref, bn1_ref,
             wn2s_ref, wn2v_ref, bn2s_ref, bn2v_ref, *rest) = refs
        else:
            (sn_ref, vn_ref, a0_ref, a1_ref, c0_ref, c1_ref, *rest) = refs
        if fuse_ln:
            (oh_ref, lnw_ref, lnb_ref, s_ref, v_ref) = rest
        else:
            (s_ref, v_ref) = rest
        sn = sn_ref[...]
        vn = vn_ref[...]
        a = a0_ref[...] + a1_ref[...]
        cnt = jnp.maximum(c0_ref[...] + c1_ref[...], 1.0)
        s_agg = a[:, 0:64]
        va = [a[:, 64 + 16 * i:80 + 16 * i] / cnt for i in range(3)]
        if use_mlp:
            vnorm = jnp.sqrt(va[0] * va[0] + va[1] * va[1] + va[2] * va[2]
                             + 1e-6)
            u = _silu(_dot_bf(sn, wn1a_ref[...]) + _dot_bf(s_agg, wn1b_ref[...])
                      + _dot_bf(vnorm, wn1c_ref[...]) + bn1_ref[...])
            s_new = sn + _dot_bf(u, wn2s_ref[...]) + bn2s_ref[...]
            uv = _dot_bf(u, wn2v_ref[...]) + bn2v_ref[...]
            v_new = jnp.concatenate(
                [vn[:, 16 * i:16 * i + 16] + va[i] * uv for i in range(3)],
                axis=1)
        else:
            s_new = sn + s_agg
            v_new = jnp.concatenate(
                [vn[:, 16 * i:16 * i + 16] + va[i] for i in range(3)], axis=1)
        if fuse_ln:
            s_new, v_new = _ln_math(s_new, v_new, oh_ref[...],
                                    lnw_ref[...], lnb_ref[...])
        s_ref[...] = s_new
        v_ref[...] = v_new
    return body


def _node_call(use_mlp, sn, vn, acc, cacc, w, ln_next=None):
    ins = [sn, vn, acc[0], acc[1], cacc[0], cacc[1]]
    if use_mlp:
        ins += [w['wn1a'], w['wn1b'], w['wn1c'], w['bn1'],
                w['wn2s'], w['wn2v'], w['bn2s'], w['bn2v']]
    fuse_ln = ln_next is not None
    if fuse_ln:
        oh, lnw, lnb = ln_next
        ins += [oh, lnw.reshape(1, _SDIM), lnb.reshape(1, _SDIM)]
    return pl.pallas_call(
        _make_node_body(use_mlp, fuse_ln),
        out_shape=[jax.ShapeDtypeStruct((_N, _SDIM), jnp.float32),
                   jax.ShapeDtypeStruct((_N, 3 * _VDIM), jnp.float32)],
    )(*ins)


@functools.lru_cache(maxsize=None)
def _make_multigather(ne, ds, tabns):
    nt = len(ds)
    cpw = ne // _NW
    c = min(cpw, 256)
    nch = cpw // c
    nb = min(2, nch)
    mesh = plsc.VectorSubcoreMesh(core_axis_name="c", subcore_axis_name="s")
    out_type = [jax.ShapeDtypeStruct((ne, d), jnp.float32) for d in ds]
    scratch = ([pltpu.VMEM((nb, c), jnp.int32) for _ in range(nt)]
               + [pltpu.VMEM((nb, c, d), jnp.float32) for d in ds]
               + [pltpu.SemaphoreType.DMA for _ in range(nt * nb)]
               + [pltpu.SemaphoreType.DMA for _ in range(nt * nb)])

    @functools.partial(pl.kernel, out_type=out_type, mesh=mesh,
                       scratch_types=scratch,
                       compiler_params=pltpu.CompilerParams(
                           use_tc_tiling_on_sc=False))
    def k(*refs):
        tabs = refs[0:nt]
        idxs = refs[nt:2 * nt]
        outs = refs[2 * nt:3 * nt]
        idx_v = refs[3 * nt:4 * nt]
        row_v = refs[4 * nt:5 * nt]
        gsem = refs[5 * nt:5 * nt + nt * nb]
        wsem = refs[5 * nt + nt * nb:5 * nt + 2 * nt * nb]
        wid = lax.axis_index("s") * 2 + lax.axis_index("c")
        base0 = wid * cpw
        gd, wd = {}, {}

        def start(ci):
            p = ci % nb
            base = base0 + ci * c
            for t in range(nt):
                pltpu.sync_copy(idxs[t].at[pl.ds(base, c)], idx_v[t].at[p])
            gd[ci] = [pltpu.async_copy(tabs[t].at[idx_v[t].at[p]],
                                       row_v[t].at[p], gsem[t * nb + p])
                      for t in range(nt)]

        def drain(ci):
            p = ci % nb
            base = base0 + ci * c
            for cp in gd[ci]:
                cp.wait()
            wd[ci] = [pltpu.async_copy(row_v[t].at[p],
                                       outs[t].at[pl.ds(base, c)],
                                       wsem[t * nb + p])
                      for t in range(nt)]

        for ci in range(nch):
            if ci >= nb:
                for cp in wd[ci - nb]:
                    cp.wait()
            start(ci)
            if ci >= 1:
                drain(ci - 1)
        drain(nch - 1)
        for ci in range(nch):
            if ci >= nch - nb:
                for cp in wd[ci]:
                    cp.wait()

    return k


def _gather(tables, idxs):
    ne = idxs[0].shape[0]
    ds = tuple(int(t.shape[1]) for t in tables)
    tabns = tuple(int(t.shape[0]) for t in tables)
    k = _make_multigather(ne, ds, tabns)
    return k(*tables, *idxs)


@functools.lru_cache(maxsize=None)
def _make_scatter(ne, d):
    cpw = ne // _NW
    c = min(cpw, 256)
    nch = cpw // c
    nb = min(2, nch)
    mesh = plsc.VectorSubcoreMesh(core_axis_name="c", subcore_axis_name="s")
    out_type = jax.ShapeDtypeStruct((2, _N, d), jnp.float32)
    scratch = [pltpu.VMEM((nb, c), jnp.int32),
               pltpu.VMEM((nb, c, d), jnp.float32),
               pltpu.VMEM_SHARED((_N, d), jnp.float32),
               pltpu.SemaphoreType.DMA, pltpu.SemaphoreType.DMA,
               pltpu.SemaphoreType.DMA, pltpu.SemaphoreType.DMA,
               pltpu.SemaphoreType.DMA, pltpu.SemaphoreType.DMA]

    @functools.partial(pl.kernel, out_type=out_type, mesh=mesh,
                       scratch_types=scratch,
                       compiler_params=pltpu.CompilerParams(
                           use_tc_tiling_on_sc=False))
    def k(msg_hbm, dst_hbm, zero_hbm, out_hbm, idx_v, rows_v, acc_sh, *sems):
        cid = lax.axis_index("c")
        sid = lax.axis_index("s")
        wid = sid * 2 + cid

        @pl.when(sid == 0)
        def _():
            pltpu.sync_copy(zero_hbm, acc_sh)

        plsc.subcore_barrier()
        ld, sd = {}, {}

        def load(ci):
            p = ci % nb
            base = wid * cpw + ci * c
            ld[ci] = [
                pltpu.async_copy(dst_hbm.at[pl.ds(base, c)], idx_v.at[p],
                                 sems[p]),
                pltpu.async_copy(msg_hbm.at[pl.ds(base, c)], rows_v.at[p],
                                 sems[nb + p]),
            ]

        load(0)
        for ci in range(nch):
            p = ci % nb
            for cp in ld[ci]:
                cp.wait()
            sd[ci] = [pltpu.async_copy(rows_v.at[p], acc_sh.at[idx_v.at[p]],
                                       sems[2 * nb + p], add=True)]
            if ci + 1 < nch:
                if ci >= 1:
                    for cp in sd[ci - 1]:
                        cp.wait()
                load(ci + 1)
        for ci in range(max(0, nch - 2), nch):
            for cp in sd[ci]:
                cp.wait()

        plsc.subcore_barrier()

        @pl.when(sid == 0)
        def _():
            pltpu.sync_copy(acc_sh, out_hbm.at[cid])

    return k


def _scatter(msg, dst, d):
    ne = msg.shape[0]
    zero = jnp.zeros((_N, d), jnp.float32)
    return _make_scatter(ne, d)(msg, dst, zero)


def _split_params(p, ltype, use_mlp):
    dflen = _RBF if ltype == 'r' else 1
    w1 = p['W1']
    w2 = p['W2']
    out = {
        'w1a': w1[0:64], 'w1b': w1[64:128],
        'w1c': w1[128:128 + dflen], 'w1e': w1[128 + dflen:],
        'b1': p['b1'].reshape(1, 64),
        'w2s': w2[:, 0:64], 'w2v1': w2[:, 64:80],
        'w2v2': w2[:, 80:96], 'w2e': w2[:, 96:112],
        'b2s': p['b2'][0:64].reshape(1, 64),
        'b2v1': p['b2'][64:80].reshape(1, 16),
        'b2v2': p['b2'][80:96].reshape(1, 16),
        'b2e': p['b2'][96:112].reshape(1, 16),
        'cen': jnp.linspace(0.0, _CUT, _RBF).reshape(1, _RBF),
    }
    if use_mlp:
        wn1 = p['Wn1']
        wn2 = p['Wn2']
        out.update({
            'wn1a': wn1[0:64], 'wn1b': wn1[64:128], 'wn1c': wn1[128:144],
            'bn1': p['bn1'].reshape(1, 64),
            'wn2s': wn2[:, 0:64], 'wn2v': wn2[:, 64:80],
            'bn2s': p['bn2'][0:64].reshape(1, 64),
            'bn2v': p['bn2'][64:80].reshape(1, 16),
        })
    return out


def kernel(s, v, edge_index_local, d_local, r_local, e_local,
           edge_index_global, d_global, r_global, e_global, batch, params):
    n = s.shape[0]
    el = edge_index_local.astype(jnp.int32)
    eg = edge_index_global.astype(jnp.int32)
    nl, ng = el.shape[1], eg.shape[1]
    src_l, dst_l = el[0], el[1]
    src_g, dst_g = eg[0], eg[1]

    kl = src_l * n + dst_l
    kg = src_g * n + dst_g
    map_g = jnp.full((n * n,), -1, jnp.int32).at[kg].max(
        jnp.arange(ng, dtype=jnp.int32))
    map_l = jnp.full((n * n,), -1, jnp.int32).at[kl].max(
        jnp.arange(nl, dtype=jnp.int32))
    g2l = map_g[kl]
    g2g = map_g[kg]
    l2g = map_l[kg]
    l2l = map_l[kl]
    idx_l1 = jnp.where(g2l >= 0, g2l, ng)
    idx_g3 = jnp.where(l2g >= 0, l2g, nl + g2g)
    idx_l4 = jnp.where(g2l >= 0, g2l, ng + l2l)

    oh = (batch[:, None] == jnp.arange(_NG)[None, :]).astype(jnp.float32)
    vflat = v.reshape(n, 3 * _VDIM)
    d_l2 = d_local[:, None]
    d_g2 = d_global[:, None]
    wts = [_split_params(params[i], _TYPES[i], _USE_MLP[i]) for i in range(5)]

    cacc_l = _scatter(jnp.ones((nl, 16), jnp.float32), dst_l, 16)
    cacc_g = _scatter(jnp.ones((ng, 16), jnp.float32), dst_g, 16)

    def run_layer(i, sn, vn, ein, etab, eidx):
        is_g = _TYPES[i] == 'g'
        src, dst = (src_g, dst_g) if is_g else (src_l, dst_l)
        d2, r = (d_g2, r_global) if is_g else (d_l2, r_local)
        ne = ng if is_g else nl
        cacc = cacc_g if is_g else cacc_l
        tabs = [sn, sn] + ([vn] if _HAS_V[i] else [])
        idxs = [src, dst] + ([src] if _HAS_V[i] else [])
        if etab is not None:
            tabs.append(etab)
            idxs.append(eidx)
        g = _gather(tuple(tabs), tuple(idxs))
        ssrc, sdst = g[0], g[1]
        vsrc = g[2] if _HAS_V[i] else None
        if etab is not None:
            ein = g[-1]
        msg, enew = _conv_call(_TYPES[i], _HAS_V[i], ne,
                               ssrc, sdst, vsrc, d2, r, ein, wts[i])
        acc = _scatter(msg, dst, 112)
        ln_next = ((oh, params[i + 1]['ln_w'], params[i + 1]['ln_b'])
                   if i < 4 else None)
        s2, v2 = _node_call(_USE_MLP[i], sn, vn, acc, cacc, wts[i], ln_next)
        return s2, v2, enew

    zrow = jnp.zeros((1, _EDIM), jnp.float32)
    sn0, vn0 = _ln_call(s, vflat, oh, params[0]['ln_w'], params[0]['ln_b'])
    s0, v0, eg0 = run_layer(0, sn0, vn0, e_global, None, None)
    eg0r = _bf(eg0)
    s1, v1, el1 = run_layer(1, s0, v0, None,
                            jnp.concatenate([eg0r, zrow], 0), idx_l1)
    s2, v2, el2 = run_layer(2, s1, v1, el1, None, None)
    el2r = _bf(el2)
    s3, v3, eg3 = run_layer(3, s2, v2, None,
                            jnp.concatenate([el2r, eg0r], 0), idx_g3)
    eg3r = _bf(eg3)
    s4, v4, el4 = run_layer(4, s3, v3, None,
                            jnp.concatenate([eg3r, el2r], 0), idx_l4)
    (e_out,) = _gather((jnp.concatenate([_bf(el4), eg3r], 0),), (idx_g3,))
    return s4, v4.reshape(n, 3, _VDIM), e_out
